# baseline (speedup 1.0000x reference)
import functools

import jax
import jax.numpy as jnp
import numpy as np
from jax import lax
from jax.experimental import pallas as pl
from jax.experimental.pallas import tpu as pltpu
from jax.experimental.pallas import tpu_sc as plsc

_B = 1024
_D = 64
_C = 16
_K = 25
_MIN_CONF = 0.25
_BIG = 3.0e38

_NW = 32
_RPW = _B // _NW
_L = 16
_NCH = _B // _L


def _d2_body(enc_ref, d2_ref):
    enc = enc_ref[...]
    sq = jnp.sum(enc * enc, axis=1)
    g = jnp.dot(enc, enc.T, preferred_element_type=jnp.float32)
    d2_ref[...] = jnp.maximum(sq[:, None] + sq[None, :] - 2.0 * g, 0.0)


def _xlane(v, op):
    iota = lax.broadcasted_iota(jnp.int32, (_L,), 0)
    for s in (1, 2, 4, 8):
        v = op(v, v[jnp.bitwise_xor(iota, s)])
    return v


def _select_body(d2_hbm, out_hbm, rows_v, thr_v):
    wid = lax.axis_index("s") * 2 + lax.axis_index("c")
    base = wid * _RPW
    pltpu.sync_copy(d2_hbm.at[pl.ds(base, _RPW)], rows_v)

    def per_row(r, _):
        def per_round(_, carry):
            m, thr = carry

            def per_chunk(c, acc):
                acc_min, acc_na = acc
                x = rows_v[r, pl.ds(c * _L, _L)]
                gt = x > m
                acc_min = jnp.minimum(acc_min, jnp.where(gt, x, _BIG))
                acc_na = acc_na + jnp.where(gt, 1, 0)
                return acc_min, acc_na

            acc0 = (jnp.full((_L,), _BIG, jnp.float32),
                    jnp.zeros((_L,), jnp.int32))
            acc_min, acc_na = lax.fori_loop(0, _NCH, per_chunk, acc0)
            mn = _xlane(acc_min, jnp.minimum)
            na = _xlane(acc_na, jnp.add)
            thr = jnp.where(_B - na <= _K, mn, thr)
            return mn, thr

        init = jnp.full((_L,), -1.0, jnp.float32)
        _, thr = lax.fori_loop(0, _K + 1, per_round, (init, init))
        thr_v[r, :] = thr
        return 0

    lax.fori_loop(0, _RPW, per_row, 0)
    pltpu.sync_copy(thr_v, out_hbm.at[pl.ds(base, _RPW)])


def _finish_body(d2_ref, thr_ref, cat_ref, ent_ref, ncomp_ref):
    d2 = d2_ref[...]
    cat = cat_ref[...]
    thresh = jnp.sqrt(thr_ref[...])

    mask = (jnp.sqrt(d2) < thresh).astype(jnp.float32)
    counts = jnp.sum(mask, axis=1)

    cidx = lax.broadcasted_iota(jnp.int32, (_B, _C), 1)
    maxg = jnp.max(cat, axis=1)
    hard = jnp.min(jnp.where(cat == maxg[:, None], cidx, _C), axis=1)
    onehot = (cidx == hard[:, None]).astype(jnp.float32)

    bins = jnp.dot(mask, onehot, preferred_element_type=jnp.float32)
    bins = bins / counts[:, None]
    ent = -jnp.sum(bins * jnp.log(bins + 1e-5), axis=1)
    ent_ref[...] = ent[:, None]

    conf = (maxg >= _MIN_CONF).astype(jnp.float32)
    populated = jnp.sum(onehot * conf[:, None], axis=0)
    ncomp_ref[...] = jnp.sum((populated > 0.0).astype(jnp.float32)).reshape(1, 1)


@functools.partial(
    pl.kernel,
    out_type=jax.ShapeDtypeStruct((_B, _L), jnp.float32),
    mesh=plsc.VectorSubcoreMesh(core_axis_name="c", subcore_axis_name="s"),
    scratch_types=[
        pltpu.VMEM((_RPW, _B), jnp.float32),
        pltpu.VMEM((_RPW, _L), jnp.float32),
    ],
)
def _select_kernel(d2_hbm, out_hbm, rows_v, thr_v):
    _select_body(d2_hbm, out_hbm, rows_v, thr_v)


def kernel(encodings, categorical):
    d2 = pl.pallas_call(
        _d2_body,
        out_shape=jax.ShapeDtypeStruct((_B, _B), jnp.float32),
    )(encodings)
    thr2 = _select_kernel(d2)[:, 0]
    ent, ncomp = pl.pallas_call(
        _finish_body,
        out_shape=[
            jax.ShapeDtypeStruct((_B, 1), jnp.float32),
            jax.ShapeDtypeStruct((1, 1), jnp.float32),
        ],
    )(d2, thr2.reshape(_B, 1), categorical)
    return encodings, ent.reshape(_B), ncomp.reshape(())


# speedup vs baseline: 7.0423x; 7.0423x over previous
import jax
import jax.numpy as jnp
import numpy as np
from jax.experimental import pallas as pl

_B = 1024
_D = 64
_C = 16
_K = 25
_MIN_CONF = 0.25
_BIG = 3.0e38
_BIG_BITS = int(np.float32(_BIG).view(np.int32))


def _overlap_body(enc_ref, cat_ref, ent_ref, ncomp_ref):
    enc = enc_ref[...]
    cat = cat_ref[...]

    sq = jnp.sum(enc * enc, axis=1)
    g = jnp.dot(enc, enc.T, preferred_element_type=jnp.float32)
    d2 = jnp.maximum(sq[:, None] + sq[None, :] - 2.0 * g, 0.0)

    def next_distinct(_, carry):
        m, thr = carry
        gt = d2 > m[:, None]
        mn = jnp.min(jnp.where(gt, d2, _BIG), axis=1)
        n_above = jnp.sum(gt.astype(jnp.float32), axis=1)
        thr = jnp.where(float(_B) - n_above <= float(_K), mn, thr)
        return mn, thr

    minus_one = jnp.full((_B,), -1.0, jnp.float32)
    _, thresh2 = jax.lax.fori_loop(
        0, _K + 1, next_distinct, (minus_one, minus_one)
    )

    thresh = jnp.sqrt(thresh2)
    dist = jnp.sqrt(d2)
    mask = (dist < thresh[:, None]).astype(jnp.float32)
    counts = jnp.sum(mask, axis=1)

    cidx = jax.lax.broadcasted_iota(jnp.int32, (_B, _C), 1)
    maxg = jnp.max(cat, axis=1)
    hard = jnp.min(jnp.where(cat == maxg[:, None], cidx, _C), axis=1)
    onehot = (cidx == hard[:, None]).astype(jnp.float32)

    bins = jnp.dot(mask, onehot, preferred_element_type=jnp.float32)
    bins = bins / counts[:, None]
    ent = -jnp.sum(bins * jnp.log(bins + 1e-5), axis=1)
    ent_ref[...] = ent[:, None]

    conf = (maxg >= _MIN_CONF).astype(jnp.float32)
    populated = jnp.sum(onehot * conf[:, None], axis=0)
    ncomp_ref[...] = jnp.sum((populated > 0.0).astype(jnp.float32)).reshape(1, 1)


def kernel(encodings, categorical):
    ent, ncomp = pl.pallas_call(
        _overlap_body,
        out_shape=[
            jax.ShapeDtypeStruct((_B, 1), jnp.float32),
            jax.ShapeDtypeStruct((1, 1), jnp.float32),
        ],
    )(encodings, categorical)
    return encodings, ent.reshape(_B), ncomp.reshape(())


# speedup vs baseline: 7.2511x; 1.0297x over previous
import jax
import jax.numpy as jnp
import numpy as np
from jax.experimental import pallas as pl

_B = 1024
_D = 64
_C = 16
_K = 25
_MIN_CONF = 0.25
_BIG = 3.0e38
_BIG_BITS = int(np.float32(_BIG).view(np.int32))
_BISECT_ROUNDS = 14


def _overlap_body(enc_ref, cat_ref, ent_ref, ncomp_ref):
    enc = enc_ref[...]
    cat = cat_ref[...]

    sq = jnp.sum(enc * enc, axis=1)
    g = jnp.dot(enc, enc.T, preferred_element_type=jnp.float32)
    d2 = jnp.maximum(sq[:, None] + sq[None, :] - 2.0 * g, 0.0)

    bits = d2.view(jnp.int32)

    def bisect(_, carry):
        lo, hi = carry
        mid = lo + jax.lax.shift_right_logical(hi - lo + 1, 1)
        cnt = jnp.sum((bits < mid[:, None]).astype(jnp.int32), axis=1)
        ok = cnt <= _K
        return jnp.where(ok, mid, lo), jnp.where(ok, hi, mid - 1)

    hi0 = jnp.full((_B,), _BIG_BITS, jnp.int32)
    lo, _ = jax.lax.fori_loop(
        0, _BISECT_ROUNDS, bisect, (jnp.zeros((_B,), jnp.int32), hi0)
    )
    m0 = lo.view(jnp.float32)

    def walk_cond(carry):
        return carry[2]

    def walk_body(carry):
        m, thr, _ = carry
        gt = d2 > m[:, None]
        mn = jnp.min(jnp.where(gt, d2, _BIG), axis=1)
        n_above = jnp.sum(gt.astype(jnp.float32), axis=1)
        live = float(_B) - n_above <= float(_K)
        thr = jnp.where(live, mn, thr)
        return mn, thr, jnp.any(live)

    _, thresh2, _ = jax.lax.while_loop(
        walk_cond, walk_body, (m0, m0, jnp.bool_(True))
    )

    thresh = jnp.sqrt(thresh2)
    dist = jnp.sqrt(d2)
    mask = (dist < thresh[:, None]).astype(jnp.float32)
    counts = jnp.sum(mask, axis=1)

    cidx = jax.lax.broadcasted_iota(jnp.int32, (_B, _C), 1)
    maxg = jnp.max(cat, axis=1)
    hard = jnp.min(jnp.where(cat == maxg[:, None], cidx, _C), axis=1)
    onehot = (cidx == hard[:, None]).astype(jnp.float32)

    bins = jnp.dot(mask, onehot, preferred_element_type=jnp.float32)
    bins = bins / counts[:, None]
    ent = -jnp.sum(bins * jnp.log(bins + 1e-5), axis=1)
    ent_ref[...] = ent[:, None]

    conf = (maxg >= _MIN_CONF).astype(jnp.float32)
    populated = jnp.sum(onehot * conf[:, None], axis=0)
    ncomp_ref[...] = jnp.sum((populated > 0.0).astype(jnp.float32)).reshape(1, 1)


def kernel(encodings, categorical):
    ent, ncomp = pl.pallas_call(
        _overlap_body,
        out_shape=[
            jax.ShapeDtypeStruct((_B, 1), jnp.float32),
            jax.ShapeDtypeStruct((1, 1), jnp.float32),
        ],
    )(encodings, categorical)
    return encodings, ent.reshape(_B), ncomp.reshape(())


# speedup vs baseline: 7.6275x; 1.0519x over previous
import jax
import jax.numpy as jnp
import numpy as np
from jax.experimental import pallas as pl

_B = 1024
_D = 64
_C = 16
_K = 25
_MIN_CONF = 0.25
_BIG = 3.0e38
_BIG_BITS = int(np.float32(_BIG).view(np.int32))
_BISECT_ROUNDS = 18


def _overlap_body(enc_ref, cat_ref, ent_ref, ncomp_ref):
    enc = enc_ref[...]
    cat = cat_ref[...]

    sq = jnp.sum(enc * enc, axis=1)
    g = jnp.dot(enc, enc.T, preferred_element_type=jnp.float32)
    d2 = jnp.maximum(sq[:, None] + sq[None, :] - 2.0 * g, 0.0)

    bits = d2.view(jnp.int32)

    def bisect(_, carry):
        lo, hi = carry
        mid = lo + jax.lax.shift_right_logical(hi - lo + 1, 1)
        cnt = jnp.sum((bits < mid[:, None]).astype(jnp.int32), axis=1)
        ok = cnt <= _K
        return jnp.where(ok, mid, lo), jnp.where(ok, hi, mid - 1)

    hi0 = jnp.full((_B,), _BIG_BITS, jnp.int32)
    lo, _ = jax.lax.fori_loop(
        0, _BISECT_ROUNDS, bisect, (jnp.zeros((_B,), jnp.int32), hi0)
    )
    m0 = lo.view(jnp.float32)

    def walk_cond(carry):
        return carry[2]

    def walk_body(carry):
        m, thr, _ = carry
        gt = d2 > m[:, None]
        mn = jnp.min(jnp.where(gt, d2, _BIG), axis=1)
        n_above = jnp.sum(gt.astype(jnp.float32), axis=1)
        live = float(_B) - n_above <= float(_K)
        thr = jnp.where(live, mn, thr)
        return mn, thr, jnp.any(live)

    _, thresh2, _ = jax.lax.while_loop(
        walk_cond, walk_body, (m0, m0, jnp.bool_(True))
    )

    thresh = jnp.sqrt(thresh2)
    dist = jnp.sqrt(d2)
    mask = (dist < thresh[:, None]).astype(jnp.float32)
    counts = jnp.sum(mask, axis=1)

    cidx = jax.lax.broadcasted_iota(jnp.int32, (_B, _C), 1)
    maxg = jnp.max(cat, axis=1)
    hard = jnp.min(jnp.where(cat == maxg[:, None], cidx, _C), axis=1)
    onehot = (cidx == hard[:, None]).astype(jnp.float32)

    bins = jnp.dot(mask, onehot, preferred_element_type=jnp.float32)
    bins = bins / counts[:, None]
    ent = -jnp.sum(bins * jnp.log(bins + 1e-5), axis=1)
    ent_ref[...] = ent[:, None]

    conf = (maxg >= _MIN_CONF).astype(jnp.float32)
    populated = jnp.sum(onehot * conf[:, None], axis=0)
    ncomp_ref[...] = jnp.sum((populated > 0.0).astype(jnp.float32)).reshape(1, 1)


def kernel(encodings, categorical):
    ent, ncomp = pl.pallas_call(
        _overlap_body,
        out_shape=[
            jax.ShapeDtypeStruct((_B, 1), jnp.float32),
            jax.ShapeDtypeStruct((1, 1), jnp.float32),
        ],
    )(encodings, categorical)
    return encodings, ent.reshape(_B), ncomp.reshape(())
